# baseline (speedup 1.0000x reference)
import functools

import jax
import jax.numpy as jnp
import numpy as np
from jax import lax
from jax.experimental import pallas as pl
from jax.experimental.pallas import tpu as pltpu
from jax.experimental.pallas import tpu_sc as plsc

_SHARED_PAIRS = [(2, 1), (4, 2), (6, 3)]
_TOPK = 4
_N = 256
_BETA = 0.5

_NC = 2
_NS = 16
_NW = _NC * _NS
_LANES = 16
_ROWS_PER_W = _N // _NW
_SLOTS = 8
_GROWS = _ROWS_PER_W * _SLOTS


@functools.cache
def _perms(P):
    with jax.ensure_compile_time_eval():
        pk1, pk2 = jax.random.split(jax.random.key(42))
        rp = np.asarray(jax.random.permutation(pk1, P)[:_N]).astype(np.int32)
        sp = np.asarray(jax.random.permutation(pk2, P)[:_N]).astype(np.int32)
    return rp, sp


@functools.cache
def _gather_ids(B, V, P):
    rp, sp = _perms(P)
    VS = V // 2
    t_ids = np.empty((_N, 2, 4), dtype=np.int32)
    s_ids = np.empty((_N, 2, 4), dtype=np.int32)
    for b in range(2):
        t_ids[:, b, 0] = (b * V + 0) * P + rp
        s_ids[:, b, 0] = (b * VS + 0) * P + rp
        for k, (tv, sv) in enumerate(_SHARED_PAIRS, start=1):
            t_ids[:, b, k] = (b * V + tv) * P + sp
            s_ids[:, b, k] = (b * VS + sv) * P + sp
    return t_ids.reshape(-1), s_ids.reshape(-1)


def _huber(d):
    ad = jnp.abs(d)
    return jnp.where(ad < _BETA, d * d, ad - 0.5 * _BETA)


def _make_sc_kernel(D):
    n_chunks = D // _LANES
    mesh = plsc.VectorSubcoreMesh(core_axis_name="c", subcore_axis_name="s")

    @functools.partial(
        pl.kernel,
        mesh=mesh,
        out_type=jax.ShapeDtypeStruct((_NW, _LANES), jnp.float32),
        scratch_types=[
            pltpu.VMEM((_GROWS,), jnp.int32),
            pltpu.VMEM((_GROWS,), jnp.int32),
            pltpu.VMEM((_GROWS, D), jnp.float32),
            pltpu.VMEM((_GROWS, D), jnp.float32),
            pltpu.VMEM((_LANES,), jnp.float32),
            pltpu.SemaphoreType.DMA,
            pltpu.SemaphoreType.DMA,
            pltpu.SemaphoreType.DMA,
            pltpu.SemaphoreType.DMA,
            pltpu.SemaphoreType.DMA,
            pltpu.SemaphoreType.DMA,
            pltpu.SemaphoreType.DMA,
            pltpu.SemaphoreType.DMA,
            pltpu.SemaphoreType.DMA,
            pltpu.SemaphoreType.DMA,
        ],
    )
    def sc_kernel(tidx_hbm, sidx_hbm, t_hbm, s_hbm, out_hbm,
                  tidx_v, sidx_v, t_v, s_v, out_v, sem_t, sem_s,
                  sem_t0, sem_t1, sem_t2, sem_t3,
                  sem_s0, sem_s1, sem_s2, sem_s3):
        sems_t = (sem_t0, sem_t1, sem_t2, sem_t3)
        sems_s = (sem_s0, sem_s1, sem_s2, sem_s3)
        wid = lax.axis_index("s") * _NC + lax.axis_index("c")
        base = wid * _GROWS
        cpi_t = pltpu.async_copy(
            tidx_hbm.at[pl.ds(base, _GROWS)], tidx_v, sem_t)
        cpi_s = pltpu.async_copy(
            sidx_hbm.at[pl.ds(base, _GROWS)], sidx_v, sem_s)
        cpi_t.wait()
        cpi_s.wait()

        n_pipe = 4
        crows = _GROWS // n_pipe
        copies = []
        for g in range(n_pipe):
            sl = pl.ds(g * crows, crows)
            copies.append((
                pltpu.async_copy(t_hbm.at[tidx_v.at[sl]], t_v.at[sl],
                                 sems_t[g]),
                pltpu.async_copy(s_hbm.at[sidx_v.at[sl]], s_v.at[sl],
                                 sems_s[g]),
            ))

        zero = jnp.zeros((_LANES,), jnp.float32)
        rb_per_chunk = crows // 4

        def body(it, acc):
            a1, a2, a3 = acc
            rb = it // n_chunks
            c = it - rb * n_chunks
            off = c * _LANES
            row0 = rb * 4
            t0 = t_v[row0, pl.ds(off, _LANES)]
            s0 = s_v[row0, pl.ds(off, _LANES)]
            e0 = s0 - t0
            a2 = a2 + _huber(e0)
            for k in (1, 2, 3):
                tk = t_v[row0 + k, pl.ds(off, _LANES)]
                sk = s_v[row0 + k, pl.ds(off, _LANES)]
                ek = sk - tk
                a1 = a1 + _huber(e0 - ek)
                a3 = a3 + _huber(ek)
            return (a1, a2, a3)

        acc = (zero, zero, zero)
        for g in range(n_pipe):
            cp_t, cp_s = copies[g]
            cp_t.wait()
            cp_s.wait()
            lo = g * rb_per_chunk * n_chunks
            acc = lax.fori_loop(
                lo, lo + rb_per_chunk * n_chunks, body, acc, unroll=8)
        a1, a2, a3 = acc

        B = 2
        n_d1 = 3 * B * _N
        n_d2 = 3 * B * _N * _TOPK
        w_d1 = 1.0 / (n_d1 * D)
        w_d2 = 12.0 / (n_d2 * D)
        w_d3 = 4.0 / (n_d2 * D)
        out_v[...] = w_d1 * a1 + w_d2 * a2 + w_d3 * a3
        pltpu.sync_copy(out_v, out_hbm.at[wid])

    return sc_kernel


def kernel(teacher_feats, student_feats):
    B, V, P, D = teacher_feats.shape
    tf = jax.lax.stop_gradient(teacher_feats)

    t_ids, s_ids = _gather_ids(B, V, P)
    t_flat = tf.reshape(B * V * P, D)
    s_flat = student_feats.reshape(B * (V // 2) * P, D)

    partials = _make_sc_kernel(D)(
        jnp.asarray(t_ids), jnp.asarray(s_ids), t_flat, s_flat)
    return jnp.sum(partials)


# speedup vs baseline: 1.4054x; 1.4054x over previous
import functools

import jax
import jax.numpy as jnp
import numpy as np
from jax import lax
from jax.experimental import pallas as pl
from jax.experimental.pallas import tpu as pltpu
from jax.experimental.pallas import tpu_sc as plsc

_SHARED_PAIRS = [(2, 1), (4, 2), (6, 3)]
_TOPK = 4
_N = 256
_BETA = 0.5

_NC = 2
_NS = 16
_NW = _NC * _NS
_LANES = 16
_ROWS_PER_W = _N // _NW
_SLOTS = 8
_GROWS = _ROWS_PER_W * _SLOTS


@functools.cache
def _perms(P):
    with jax.ensure_compile_time_eval():
        pk1, pk2 = jax.random.split(jax.random.key(42))
        rp = np.asarray(jax.random.permutation(pk1, P)[:_N]).astype(np.int32)
        sp = np.asarray(jax.random.permutation(pk2, P)[:_N]).astype(np.int32)
    return rp, sp


@functools.cache
def _gather_ids(B, V, P):
    rp, sp = _perms(P)
    VS = V // 2
    t_ids = np.empty((_N, 2, 4), dtype=np.int32)
    s_ids = np.empty((_N, 2, 4), dtype=np.int32)
    for b in range(2):
        t_ids[:, b, 0] = (b * V + 0) * P + rp
        s_ids[:, b, 0] = (b * VS + 0) * P + rp
        for k, (tv, sv) in enumerate(_SHARED_PAIRS, start=1):
            t_ids[:, b, k] = (b * V + tv) * P + sp
            s_ids[:, b, k] = (b * VS + sv) * P + sp
    return t_ids.reshape(-1), s_ids.reshape(-1)


def _huber(d):
    ad = jnp.abs(d)
    return jnp.where(ad < _BETA, d * d, ad - 0.5 * _BETA)


def _make_sc_kernel(D):
    n_chunks = D // _LANES
    mesh = plsc.VectorSubcoreMesh(core_axis_name="c", subcore_axis_name="s")

    @functools.partial(
        pl.kernel,
        mesh=mesh,
        out_type=jax.ShapeDtypeStruct((_NW, _LANES), jnp.float32),
        scratch_types=[
            pltpu.VMEM((_GROWS,), jnp.int32),
            pltpu.VMEM((_GROWS,), jnp.int32),
            pltpu.VMEM((_GROWS, D), jnp.float32),
            pltpu.VMEM((_GROWS, D), jnp.float32),
            pltpu.VMEM((_LANES,), jnp.float32),
            pltpu.SemaphoreType.DMA,
            pltpu.SemaphoreType.DMA,
            pltpu.SemaphoreType.DMA,
            pltpu.SemaphoreType.DMA,
            pltpu.SemaphoreType.DMA,
            pltpu.SemaphoreType.DMA,
            pltpu.SemaphoreType.DMA,
            pltpu.SemaphoreType.DMA,
            pltpu.SemaphoreType.DMA,
            pltpu.SemaphoreType.DMA,
        ],
    )
    def sc_kernel(tidx_hbm, sidx_hbm, t_hbm, s_hbm, out_hbm,
                  tidx_v, sidx_v, t_v, s_v, out_v, sem_t, sem_s,
                  sem_t0, sem_t1, sem_t2, sem_t3,
                  sem_s0, sem_s1, sem_s2, sem_s3):
        sems_t = (sem_t0, sem_t1, sem_t2, sem_t3)
        sems_s = (sem_s0, sem_s1, sem_s2, sem_s3)
        wid = lax.axis_index("s") * _NC + lax.axis_index("c")
        base = wid * _GROWS
        cpi_t = pltpu.async_copy(
            tidx_hbm.at[pl.ds(base, _GROWS)], tidx_v, sem_t)
        cpi_s = pltpu.async_copy(
            sidx_hbm.at[pl.ds(base, _GROWS)], sidx_v, sem_s)
        cpi_t.wait()
        cpi_s.wait()

        n_pipe = 4
        crows = _GROWS // n_pipe
        copies = []
        for g in range(n_pipe):
            sl = pl.ds(g * crows, crows)
            copies.append((
                pltpu.async_copy(t_hbm.at[tidx_v.at[sl]], t_v.at[sl],
                                 sems_t[g]),
                pltpu.async_copy(s_hbm.at[sidx_v.at[sl]], s_v.at[sl],
                                 sems_s[g]),
            ))

        zero = jnp.zeros((_LANES,), jnp.float32)
        rb_per_chunk = crows // 4

        def body(it, acc):
            a1, a2, a3 = acc
            rb = it // n_chunks
            c = it - rb * n_chunks
            off = c * _LANES
            row0 = rb * 4
            t0 = t_v[row0, pl.ds(off, _LANES)]
            s0 = s_v[row0, pl.ds(off, _LANES)]
            e0 = s0 - t0
            a2 = a2 + _huber(e0)
            for k in (1, 2, 3):
                tk = t_v[row0 + k, pl.ds(off, _LANES)]
                sk = s_v[row0 + k, pl.ds(off, _LANES)]
                ek = sk - tk
                a1 = a1 + _huber(e0 - ek)
                a3 = a3 + _huber(ek)
            return (a1, a2, a3)

        acc = (zero, zero, zero)
        for g in range(n_pipe):
            cp_t, cp_s = copies[g]
            cp_t.wait()
            cp_s.wait()
            lo = g * rb_per_chunk * n_chunks
            acc = lax.fori_loop(
                lo, lo + rb_per_chunk * n_chunks, body, acc, unroll=4)
        a1, a2, a3 = acc

        B = 2
        n_d1 = 3 * B * _N
        n_d2 = 3 * B * _N * _TOPK
        w_d1 = 1.0 / (n_d1 * D)
        w_d2 = 12.0 / (n_d2 * D)
        w_d3 = 4.0 / (n_d2 * D)
        out_v[...] = w_d1 * a1 + w_d2 * a2 + w_d3 * a3
        pltpu.sync_copy(out_v, out_hbm.at[wid])

    return sc_kernel


def kernel(teacher_feats, student_feats):
    B, V, P, D = teacher_feats.shape
    tf = jax.lax.stop_gradient(teacher_feats)

    t_ids, s_ids = _gather_ids(B, V, P)
    t_flat = tf.reshape(B * V * P, D)
    s_flat = student_feats.reshape(B * (V // 2) * P, D)

    partials = _make_sc_kernel(D)(
        jnp.asarray(t_ids), jnp.asarray(s_ids), t_flat, s_flat)
    return jnp.sum(partials)
